# Initial kernel scaffold; baseline (speedup 1.0000x reference)
import jax
import jax.numpy as jnp
from jax.experimental import pallas as pl


def kernel(states, edge_index):
    raise NotImplementedError("write your pallas kernel here")


# speedup vs baseline: 18.4580x; 18.4580x over previous
import functools

import jax
import jax.numpy as jnp
from jax import lax
from jax.experimental import pallas as pl
from jax.experimental.pallas import tpu as pltpu
from jax.experimental.pallas import tpu_sc as plsc

NC = 2
NS = 16
NW = NC * NS
L = 16


def _ceil_to(x, m):
    return (x + m - 1) // m * m


@functools.lru_cache(maxsize=None)
def _make_sc_call(T, N, E):
    assert T == L, "kernel assumes one timestep per vector lane"
    CB = 1024
    EW = _ceil_to(E, NW * CB) // NW
    NCHUNK = EW // CB
    EP = EW * NW
    NP = _ceil_to(N, NW * 8)
    RW = NP // NW

    mesh = plsc.VectorSubcoreMesh(core_axis_name="c", subcore_axis_name="s")

    def body(tabu, tabv, rows, cols, outpe, outke,
             idxr, idxc, gr, gc, vbuf, osc, sem1, sem2):
        wid = lax.axis_index("s") * NC + lax.axis_index("c")
        ebase = wid * EW
        zero = jnp.zeros((L,), jnp.float32)

        def chunk(i, acc):
            base = ebase + i * CB
            pltpu.sync_copy(rows.at[pl.ds(base, CB)], idxr)
            pltpu.sync_copy(cols.at[pl.ds(base, CB)], idxc)
            cp1 = pltpu.async_copy(tabu.at[idxr], gr, sem1)
            cp2 = pltpu.async_copy(tabu.at[idxc], gc, sem2)
            cp1.wait()
            cp2.wait()

            def edge(e, a):
                du = gr[e] - gc[e]
                return a + du * du

            return acc + lax.fori_loop(0, CB, edge, zero, unroll=8)

        pe = lax.fori_loop(0, NCHUNK, chunk, zero)
        osc[...] = pe
        pltpu.sync_copy(osc, outpe.at[wid])

        pltpu.sync_copy(tabv.at[pl.ds(wid * RW, RW)], vbuf)

        def krow(r, a):
            vv = vbuf[r]
            return a + vv * vv

        ke = lax.fori_loop(0, RW, krow, zero, unroll=8)
        osc[...] = ke
        pltpu.sync_copy(osc, outke.at[wid])

    call = pl.kernel(
        body,
        out_type=(
            jax.ShapeDtypeStruct((NW, L), jnp.float32),
            jax.ShapeDtypeStruct((NW, L), jnp.float32),
        ),
        mesh=mesh,
        scratch_types=[
            pltpu.VMEM((CB,), jnp.int32),
            pltpu.VMEM((CB,), jnp.int32),
            pltpu.VMEM((CB, L), jnp.float32),
            pltpu.VMEM((CB, L), jnp.float32),
            pltpu.VMEM((RW, L), jnp.float32),
            pltpu.VMEM((L,), jnp.float32),
            pltpu.SemaphoreType.DMA,
            pltpu.SemaphoreType.DMA,
        ],
        compiler_params=pltpu.CompilerParams(use_tc_tiling_on_sc=False),
    )
    return call, EP, NP


def kernel(states, edge_index):
    T, N, _ = states.shape
    E = edge_index.shape[1]
    call, EP, NP = _make_sc_call(T, N, E)

    tabu = states[:, :, 0].T
    tabv = jnp.pad(states[:, :, 1].T, ((0, NP - N), (0, 0)))
    ei = edge_index.astype(jnp.int32)
    eip = jnp.pad(ei, ((0, 0), (0, EP - E)))
    outpe, outke = call(tabu, tabv, eip[0], eip[1])

    H = 0.5 * (jnp.sum(outpe, axis=0) + jnp.sum(outke, axis=0))
    dH = H[1:] - H[:-1]
    return jnp.sum(dH * dH) / (T - 1)
